# baseline (speedup 1.0000x reference)
import jax
import jax.numpy as jnp
from jax import lax
from jax.experimental import pallas as pl
from jax.experimental.pallas import tpu as pltpu
from jax.experimental.pallas import tpu_sc as plsc

L = 16
NS = 16
B = 4096
D = 64
BPW = B // NS
CHUNK = 128
NCH = BPW // CHUNK


def _body(i0_hbm, i1_hbm, temb_hbm, nemb_hbm, tbias_hbm, nbias_hbm, out_hbm,
          idx0_v, idx1_v,
          rows_t, rows_n, bval_t, bval_n,
          acc_v, all_v, out_v, shared,
          sem_e, sem_b):
    sid = lax.axis_index("s")
    base = sid * NCH

    pltpu.sync_copy(i0_hbm.at[pl.ds(base, NCH)], idx0_v)
    pltpu.sync_copy(i1_hbm.at[pl.ds(base, NCH)], idx1_v)

    emb_cps = []
    for k in range(NCH):
        emb_cps.append(pltpu.async_copy(
            temb_hbm.at[idx0_v.at[k]], rows_t.at[pl.ds(k * CHUNK, CHUNK)], sem_e))
        emb_cps.append(pltpu.async_copy(
            nemb_hbm.at[idx1_v.at[k]], rows_n.at[pl.ds(k * CHUNK, CHUNK)], sem_e))

    bias_cps = []
    for k in range(NCH):
        bias_cps.append(pltpu.async_copy(
            tbias_hbm.at[idx0_v.at[k]], bval_t.at[pl.ds(k * CHUNK, CHUNK)], sem_b))
        bias_cps.append(pltpu.async_copy(
            nbias_hbm.at[idx1_v.at[k]], bval_n.at[pl.ds(k * CHUNK, CHUNK)], sem_b))

    for cp in emb_cps:
        cp.wait()

    def dot_step(b, accs):
        a0, a1, a2, a3 = accs
        a0 = a0 + rows_t[b, pl.ds(0, L)] * rows_n[b, pl.ds(0, L)]
        a1 = a1 + rows_t[b, pl.ds(L, L)] * rows_n[b, pl.ds(L, L)]
        a2 = a2 + rows_t[b, pl.ds(2 * L, L)] * rows_n[b, pl.ds(2 * L, L)]
        a3 = a3 + rows_t[b, pl.ds(3 * L, L)] * rows_n[b, pl.ds(3 * L, L)]
        return a0, a1, a2, a3

    zeros = jnp.zeros((L,), jnp.float32)
    a0, a1, a2, a3 = lax.fori_loop(0, BPW, dot_step, (zeros, zeros, zeros, zeros))
    acc = (a0 + a1) + (a2 + a3)

    acc_v[...] = acc
    pltpu.sync_copy(acc_v, shared.at[sid])
    plsc.subcore_barrier()
    pltpu.sync_copy(shared, all_v)
    s = all_v[0, pl.ds(0, L)]
    for r in range(1, NS):
        s = s + all_v[r, pl.ds(0, L)]
    lanes = jnp.arange(L, dtype=jnp.int32)
    for off in (8, 4, 2, 1):
        perm = jnp.bitwise_xor(lanes, off)
        s = s + s.at[perm].get(mode="promise_in_bounds")
    big_s = s

    for cp in bias_cps:
        cp.wait()

    for j in range(BPW // L):
        sl = pl.ds(j * L, L)
        x = big_s + bval_t[sl] + bval_n[sl]
        out_v[sl] = 1.0 / (1.0 + jnp.exp(-x))

    pltpu.sync_copy(out_v, out_hbm.at[pl.ds(sid * BPW, BPW)])


def kernel(inputs, track_embedding, name_embedding, track_bias, name_bias):
    num_track = track_embedding.shape[0]
    i0 = inputs[:, 0].reshape(B // CHUNK, CHUNK)
    i1 = inputs[:, 1].reshape(B // CHUNK, CHUNK)
    tb = track_bias.reshape(num_track)
    nb = name_bias.reshape(num_track)

    mesh = plsc.VectorSubcoreMesh(
        core_axis_name="c", subcore_axis_name="s", num_cores=1)
    run = pl.kernel(
        _body,
        out_type=jax.ShapeDtypeStruct((B,), jnp.float32),
        mesh=mesh,
        compiler_params=pltpu.CompilerParams(use_tc_tiling_on_sc=False),
        scratch_types=[
            pltpu.VMEM((NCH, CHUNK), jnp.int32),
            pltpu.VMEM((NCH, CHUNK), jnp.int32),
            pltpu.VMEM((BPW, D), jnp.float32),
            pltpu.VMEM((BPW, D), jnp.float32),
            pltpu.VMEM((BPW,), jnp.float32),
            pltpu.VMEM((BPW,), jnp.float32),
            pltpu.VMEM((L,), jnp.float32),
            pltpu.VMEM((NS, L), jnp.float32),
            pltpu.VMEM((BPW,), jnp.float32),
            pltpu.VMEM_SHARED((NS, L), jnp.float32),
            pltpu.SemaphoreType.DMA,
            pltpu.SemaphoreType.DMA,
        ],
    )
    out = run(i0, i1, track_embedding, name_embedding, tb, nb)
    return out.reshape(B, 1)


# speedup vs baseline: 1.1524x; 1.1524x over previous
import functools

import jax
import jax.numpy as jnp
from jax import lax
from jax.experimental import pallas as pl
from jax.experimental.pallas import tpu as pltpu
from jax.experimental.pallas import tpu_sc as plsc

L = 16
NS = 16
B = 4096
D = 64
DP = 128
BPW = B // NS
CHUNK = 128
NCH = BPW // CHUNK
BR = 1024


def _transpose_body(tt_ref, nt_ref, o_ref):
    o_ref[...] = jnp.concatenate([tt_ref[...].T, nt_ref[...].T], axis=1)


def _relayout(tt, nt, num_rows):
    grid = (pl.cdiv(num_rows, BR),)
    return pl.pallas_call(
        _transpose_body,
        grid=grid,
        in_specs=[
            pl.BlockSpec((D, BR), lambda i: (0, i)),
            pl.BlockSpec((D, BR), lambda i: (0, i)),
        ],
        out_specs=pl.BlockSpec((BR, DP), lambda i: (i, 0)),
        out_shape=jax.ShapeDtypeStruct((num_rows, DP), jnp.float32),
    )(tt, nt)


def _body(i0_hbm, i1_hbm, comb_hbm, tbias_hbm, nbias_hbm, out_hbm,
          idx0_v, idx1_v,
          rows_t, rows_n, bval_t, bval_n,
          acc_v, all_v, out_v, shared,
          sem_e, sem_b):
    sid = lax.axis_index("s")
    base = sid * NCH

    pltpu.sync_copy(i0_hbm.at[pl.ds(base, NCH)], idx0_v)
    pltpu.sync_copy(i1_hbm.at[pl.ds(base, NCH)], idx1_v)

    emb_cps = []
    for k in range(NCH):
        emb_cps.append(pltpu.async_copy(
            comb_hbm.at[idx0_v.at[k]], rows_t.at[pl.ds(k * CHUNK, CHUNK)], sem_e))
        emb_cps.append(pltpu.async_copy(
            comb_hbm.at[idx1_v.at[k]], rows_n.at[pl.ds(k * CHUNK, CHUNK)], sem_e))

    bias_cps = []
    for k in range(NCH):
        bias_cps.append(pltpu.async_copy(
            tbias_hbm.at[idx0_v.at[k]], bval_t.at[pl.ds(k * CHUNK, CHUNK)], sem_b))
        bias_cps.append(pltpu.async_copy(
            nbias_hbm.at[idx1_v.at[k]], bval_n.at[pl.ds(k * CHUNK, CHUNK)], sem_b))

    for cp in emb_cps:
        cp.wait()

    def dot_step(b, accs):
        a0, a1, a2, a3 = accs
        a0 = a0 + rows_t[b, pl.ds(0, L)] * rows_n[b, pl.ds(D, L)]
        a1 = a1 + rows_t[b, pl.ds(L, L)] * rows_n[b, pl.ds(D + L, L)]
        a2 = a2 + rows_t[b, pl.ds(2 * L, L)] * rows_n[b, pl.ds(D + 2 * L, L)]
        a3 = a3 + rows_t[b, pl.ds(3 * L, L)] * rows_n[b, pl.ds(D + 3 * L, L)]
        return a0, a1, a2, a3

    zeros = jnp.zeros((L,), jnp.float32)
    a0, a1, a2, a3 = lax.fori_loop(0, BPW, dot_step, (zeros, zeros, zeros, zeros))
    acc = (a0 + a1) + (a2 + a3)

    acc_v[...] = acc
    pltpu.sync_copy(acc_v, shared.at[pl.ds(sid * L, L)])
    plsc.subcore_barrier()
    pltpu.sync_copy(shared, all_v)
    s = all_v[pl.ds(0, L)]
    for r in range(1, NS):
        s = s + all_v[pl.ds(r * L, L)]
    lanes = jnp.arange(L, dtype=jnp.int32)
    for off in (8, 4, 2, 1):
        perm = jnp.bitwise_xor(lanes, off)
        s = s + s.at[perm].get(mode="promise_in_bounds")
    big_s = s

    for cp in bias_cps:
        cp.wait()

    for j in range(BPW // L):
        sl = pl.ds(j * L, L)
        x = big_s + bval_t[sl] + bval_n[sl]
        out_v[sl] = 1.0 / (1.0 + jnp.exp(-x))

    pltpu.sync_copy(out_v, out_hbm.at[pl.ds(sid * BPW, BPW)])


def kernel(inputs, track_embedding, name_embedding, track_bias, name_bias):
    num_track = track_embedding.shape[0]
    i0 = inputs[:, 0].reshape(B // CHUNK, CHUNK)
    i1 = inputs[:, 1].reshape(B // CHUNK, CHUNK)
    tb = track_bias.reshape(num_track)
    nb = name_bias.reshape(num_track)

    comb = _relayout(track_embedding.T, name_embedding.T, num_track)

    mesh = plsc.VectorSubcoreMesh(
        core_axis_name="c", subcore_axis_name="s", num_cores=1)
    run = pl.kernel(
        _body,
        out_type=jax.ShapeDtypeStruct((B,), jnp.float32),
        mesh=mesh,
        compiler_params=pltpu.CompilerParams(use_tc_tiling_on_sc=True),
        scratch_types=[
            pltpu.VMEM((NCH, CHUNK), jnp.int32),
            pltpu.VMEM((NCH, CHUNK), jnp.int32),
            pltpu.VMEM((BPW, DP), jnp.float32),
            pltpu.VMEM((BPW, DP), jnp.float32),
            pltpu.VMEM((BPW,), jnp.float32),
            pltpu.VMEM((BPW,), jnp.float32),
            pltpu.VMEM((L,), jnp.float32),
            pltpu.VMEM((NS * L,), jnp.float32),
            pltpu.VMEM((BPW,), jnp.float32),
            pltpu.VMEM_SHARED((NS * L,), jnp.float32),
            pltpu.SemaphoreType.DMA,
            pltpu.SemaphoreType.DMA,
        ],
    )
    out = run(i0, i1, comb, tb, nb)
    return out.reshape(B, 1)


# speedup vs baseline: 1.4432x; 1.2523x over previous
import functools

import jax
import jax.numpy as jnp
from jax import lax
from jax.experimental import pallas as pl
from jax.experimental.pallas import tpu as pltpu
from jax.experimental.pallas import tpu_sc as plsc

L = 16
NS = 16
B = 4096
D = 64
DP = 128
BPW = B // NS
CHUNK = 128
NCH = BPW // CHUNK
BR = 2048


def _transpose_body(tt_ref, nt_ref, o_ref):
    eye = jnp.eye(D, dtype=jnp.float32)
    dims = (((0,), (0,)), ((), ()))
    o_ref[:, 0:D] = lax.dot_general(
        tt_ref[...], eye, dims, preferred_element_type=jnp.float32)
    o_ref[:, D:DP] = lax.dot_general(
        nt_ref[...], eye, dims, preferred_element_type=jnp.float32)


def _relayout(tt, nt, num_rows):
    grid = (pl.cdiv(num_rows, BR),)
    return pl.pallas_call(
        _transpose_body,
        grid=grid,
        in_specs=[
            pl.BlockSpec((D, BR), lambda i: (0, i)),
            pl.BlockSpec((D, BR), lambda i: (0, i)),
        ],
        out_specs=pl.BlockSpec((BR, DP), lambda i: (i, 0)),
        out_shape=jax.ShapeDtypeStruct((num_rows, DP), jnp.float32),
    )(tt, nt)


def _body(i0_hbm, i1_hbm, comb_hbm, tbias_hbm, nbias_hbm, out_hbm,
          idx0_v, idx1_v,
          rows_t, rows_n, bval_t, bval_n,
          acc_v, all_v, out_v, shared,
          sem_e, sem_b):
    sid = lax.axis_index("s")
    base = sid * NCH

    pltpu.sync_copy(i0_hbm.at[pl.ds(base, NCH)], idx0_v)
    pltpu.sync_copy(i1_hbm.at[pl.ds(base, NCH)], idx1_v)

    emb_cps = []
    for k in range(NCH):
        emb_cps.append(pltpu.async_copy(
            comb_hbm.at[idx0_v.at[k]], rows_t.at[pl.ds(k * CHUNK, CHUNK)], sem_e))
        emb_cps.append(pltpu.async_copy(
            comb_hbm.at[idx1_v.at[k]], rows_n.at[pl.ds(k * CHUNK, CHUNK)], sem_e))

    bias_cps = []
    for k in range(NCH):
        bias_cps.append(pltpu.async_copy(
            tbias_hbm.at[idx0_v.at[k]], bval_t.at[pl.ds(k * CHUNK, CHUNK)], sem_b))
        bias_cps.append(pltpu.async_copy(
            nbias_hbm.at[idx1_v.at[k]], bval_n.at[pl.ds(k * CHUNK, CHUNK)], sem_b))

    for cp in emb_cps:
        cp.wait()

    def dot_step(b, accs):
        a0, a1, a2, a3 = accs
        a0 = a0 + rows_t[b, pl.ds(0, L)] * rows_n[b, pl.ds(D, L)]
        a1 = a1 + rows_t[b, pl.ds(L, L)] * rows_n[b, pl.ds(D + L, L)]
        a2 = a2 + rows_t[b, pl.ds(2 * L, L)] * rows_n[b, pl.ds(D + 2 * L, L)]
        a3 = a3 + rows_t[b, pl.ds(3 * L, L)] * rows_n[b, pl.ds(D + 3 * L, L)]
        return a0, a1, a2, a3

    zeros = jnp.zeros((L,), jnp.float32)
    a0, a1, a2, a3 = lax.fori_loop(0, BPW, dot_step, (zeros, zeros, zeros, zeros))
    acc = (a0 + a1) + (a2 + a3)

    acc_v[...] = acc
    pltpu.sync_copy(acc_v, shared.at[pl.ds(sid * L, L)])
    plsc.subcore_barrier()
    pltpu.sync_copy(shared, all_v)
    s = all_v[pl.ds(0, L)]
    for r in range(1, NS):
        s = s + all_v[pl.ds(r * L, L)]
    lanes = jnp.arange(L, dtype=jnp.int32)
    for off in (8, 4, 2, 1):
        perm = jnp.bitwise_xor(lanes, off)
        s = s + s.at[perm].get(mode="promise_in_bounds")
    big_s = s

    for cp in bias_cps:
        cp.wait()

    for j in range(BPW // L):
        sl = pl.ds(j * L, L)
        x = big_s + bval_t[sl] + bval_n[sl]
        out_v[sl] = 1.0 / (1.0 + jnp.exp(-x))

    pltpu.sync_copy(out_v, out_hbm.at[pl.ds(sid * BPW, BPW)])


def kernel(inputs, track_embedding, name_embedding, track_bias, name_bias):
    num_track = track_embedding.shape[0]
    i0 = inputs[:, 0].reshape(B // CHUNK, CHUNK)
    i1 = inputs[:, 1].reshape(B // CHUNK, CHUNK)
    tb = track_bias.reshape(num_track)
    nb = name_bias.reshape(num_track)

    comb = _relayout(track_embedding.T, name_embedding.T, num_track)

    mesh = plsc.VectorSubcoreMesh(
        core_axis_name="c", subcore_axis_name="s", num_cores=1)
    run = pl.kernel(
        _body,
        out_type=jax.ShapeDtypeStruct((B,), jnp.float32),
        mesh=mesh,
        compiler_params=pltpu.CompilerParams(use_tc_tiling_on_sc=True),
        scratch_types=[
            pltpu.VMEM((NCH, CHUNK), jnp.int32),
            pltpu.VMEM((NCH, CHUNK), jnp.int32),
            pltpu.VMEM((BPW, DP), jnp.float32),
            pltpu.VMEM((BPW, DP), jnp.float32),
            pltpu.VMEM((BPW,), jnp.float32),
            pltpu.VMEM((BPW,), jnp.float32),
            pltpu.VMEM((L,), jnp.float32),
            pltpu.VMEM((NS * L,), jnp.float32),
            pltpu.VMEM((BPW,), jnp.float32),
            pltpu.VMEM_SHARED((NS * L,), jnp.float32),
            pltpu.SemaphoreType.DMA,
            pltpu.SemaphoreType.DMA,
        ],
    )
    out = run(i0, i1, comb, tb, nb)
    return out.reshape(B, 1)


# speedup vs baseline: 1.7670x; 1.2244x over previous
import functools

import jax
import jax.numpy as jnp
from jax import lax
from jax.experimental import pallas as pl
from jax.experimental.pallas import tpu as pltpu
from jax.experimental.pallas import tpu_sc as plsc

L = 16
NS = 16
B = 4096
D = 64
DP = 128
BPW = B // NS
CHUNK = 128
NCH = BPW // CHUNK
BR = 4096


def _transpose_body(tt_ref, nt_ref, tb_ref, nb_ref, o_ref, tbo_ref, nbo_ref):
    eye = jnp.eye(D, dtype=jnp.float32)
    dims = (((0,), (0,)), ((), ()))
    o_ref[:, 0:D] = lax.dot_general(
        tt_ref[...], eye, dims, preferred_element_type=jnp.float32)
    o_ref[:, D:DP] = lax.dot_general(
        nt_ref[...], eye, dims, preferred_element_type=jnp.float32)
    tbo_ref[...] = tb_ref[0, :]
    nbo_ref[...] = nb_ref[0, :]


def _relayout(tt, nt, tb2, nb2, num_rows):
    grid = (pl.cdiv(num_rows, BR),)
    return pl.pallas_call(
        _transpose_body,
        grid=grid,
        in_specs=[
            pl.BlockSpec((D, BR), lambda i: (0, i)),
            pl.BlockSpec((D, BR), lambda i: (0, i)),
            pl.BlockSpec((1, BR), lambda i: (0, i)),
            pl.BlockSpec((1, BR), lambda i: (0, i)),
        ],
        out_specs=[
            pl.BlockSpec((BR, DP), lambda i: (i, 0)),
            pl.BlockSpec((BR,), lambda i: (i,)),
            pl.BlockSpec((BR,), lambda i: (i,)),
        ],
        out_shape=[
            jax.ShapeDtypeStruct((num_rows, DP), jnp.float32),
            jax.ShapeDtypeStruct((num_rows,), jnp.float32),
            jax.ShapeDtypeStruct((num_rows,), jnp.float32),
        ],
        compiler_params=pltpu.CompilerParams(
            fuse_transposed_lhs_in_matmul=True),
    )(tt, nt, tb2, nb2)


def _body(i0_hbm, i1_hbm, comb_hbm, tbias_hbm, nbias_hbm, out_hbm,
          idx0_v, idx1_v,
          rows_t, rows_n, bval_t, bval_n,
          acc_v, all_v, out_v, shared,
          sem_e, sem_b):
    sid = lax.axis_index("s")
    base = sid * NCH

    pltpu.sync_copy(i0_hbm.at[pl.ds(base, NCH)], idx0_v)
    pltpu.sync_copy(i1_hbm.at[pl.ds(base, NCH)], idx1_v)

    emb_cps = []
    for k in range(NCH):
        emb_cps.append(pltpu.async_copy(
            comb_hbm.at[idx0_v.at[k]], rows_t.at[pl.ds(k * CHUNK, CHUNK)], sem_e))
        emb_cps.append(pltpu.async_copy(
            comb_hbm.at[idx1_v.at[k]], rows_n.at[pl.ds(k * CHUNK, CHUNK)], sem_e))

    bias_cps = []
    for k in range(NCH):
        bias_cps.append(pltpu.async_copy(
            tbias_hbm.at[idx0_v.at[k]], bval_t.at[pl.ds(k * CHUNK, CHUNK)], sem_b))
        bias_cps.append(pltpu.async_copy(
            nbias_hbm.at[idx1_v.at[k]], bval_n.at[pl.ds(k * CHUNK, CHUNK)], sem_b))

    for cp in emb_cps:
        cp.wait()

    def dot_step(b, accs):
        a0, a1, a2, a3 = accs
        a0 = a0 + rows_t[b, pl.ds(0, L)] * rows_n[b, pl.ds(D, L)]
        a1 = a1 + rows_t[b, pl.ds(L, L)] * rows_n[b, pl.ds(D + L, L)]
        a2 = a2 + rows_t[b, pl.ds(2 * L, L)] * rows_n[b, pl.ds(D + 2 * L, L)]
        a3 = a3 + rows_t[b, pl.ds(3 * L, L)] * rows_n[b, pl.ds(D + 3 * L, L)]
        return a0, a1, a2, a3

    zeros = jnp.zeros((L,), jnp.float32)
    a0, a1, a2, a3 = lax.fori_loop(0, BPW, dot_step, (zeros, zeros, zeros, zeros))
    acc = (a0 + a1) + (a2 + a3)

    acc_v[...] = acc
    pltpu.sync_copy(acc_v, shared.at[pl.ds(sid * L, L)])
    plsc.subcore_barrier()
    pltpu.sync_copy(shared, all_v)
    s = all_v[pl.ds(0, L)]
    for r in range(1, NS):
        s = s + all_v[pl.ds(r * L, L)]
    lanes = jnp.arange(L, dtype=jnp.int32)
    for off in (8, 4, 2, 1):
        perm = jnp.bitwise_xor(lanes, off)
        s = s + s.at[perm].get(mode="promise_in_bounds")
    big_s = s

    for cp in bias_cps:
        cp.wait()

    for j in range(BPW // L):
        sl = pl.ds(j * L, L)
        x = big_s + bval_t[sl] + bval_n[sl]
        out_v[sl] = 1.0 / (1.0 + jnp.exp(-x))

    pltpu.sync_copy(out_v, out_hbm.at[pl.ds(sid * BPW, BPW)])


def kernel(inputs, track_embedding, name_embedding, track_bias, name_bias):
    num_track = track_embedding.shape[0]
    i0 = inputs[:, 0].reshape(B // CHUNK, CHUNK)
    i1 = inputs[:, 1].reshape(B // CHUNK, CHUNK)
    comb, tb, nb = _relayout(
        track_embedding.T, name_embedding.T,
        track_bias.T, name_bias.T, num_track)

    mesh = plsc.VectorSubcoreMesh(
        core_axis_name="c", subcore_axis_name="s", num_cores=1)
    run = pl.kernel(
        _body,
        out_type=jax.ShapeDtypeStruct((B,), jnp.float32),
        mesh=mesh,
        compiler_params=pltpu.CompilerParams(use_tc_tiling_on_sc=True),
        scratch_types=[
            pltpu.VMEM((NCH, CHUNK), jnp.int32),
            pltpu.VMEM((NCH, CHUNK), jnp.int32),
            pltpu.VMEM((BPW, DP), jnp.float32),
            pltpu.VMEM((BPW, DP), jnp.float32),
            pltpu.VMEM((BPW,), jnp.float32),
            pltpu.VMEM((BPW,), jnp.float32),
            pltpu.VMEM((L,), jnp.float32),
            pltpu.VMEM((NS * L,), jnp.float32),
            pltpu.VMEM((BPW,), jnp.float32),
            pltpu.VMEM_SHARED((NS * L,), jnp.float32),
            pltpu.SemaphoreType.DMA,
            pltpu.SemaphoreType.DMA,
        ],
    )
    out = run(i0, i1, comb, tb, nb)
    return out.reshape(B, 1)


# speedup vs baseline: 1.7754x; 1.0048x over previous
import functools

import jax
import jax.numpy as jnp
from jax import lax
from jax.experimental import pallas as pl
from jax.experimental.pallas import tpu as pltpu
from jax.experimental.pallas import tpu_sc as plsc

L = 16
NS = 16
B = 4096
D = 64
DP = 128
BPW = B // NS
CHUNK = 128
NCH = BPW // CHUNK
BR = 4096


def _transpose_body(tt_ref, nt_ref, tb_ref, nb_ref, o_ref, tbo_ref, nbo_ref):
    o_ref[:, 0:D] = tt_ref[...].T
    o_ref[:, D:DP] = nt_ref[...].T
    tbo_ref[...] = tb_ref[0, :]
    nbo_ref[...] = nb_ref[0, :]


def _relayout(tt, nt, tb2, nb2, num_rows):
    grid = (pl.cdiv(num_rows, BR),)
    return pl.pallas_call(
        _transpose_body,
        grid=grid,
        in_specs=[
            pl.BlockSpec((D, BR), lambda i: (0, i)),
            pl.BlockSpec((D, BR), lambda i: (0, i)),
            pl.BlockSpec((1, BR), lambda i: (0, i)),
            pl.BlockSpec((1, BR), lambda i: (0, i)),
        ],
        out_specs=[
            pl.BlockSpec((BR, DP), lambda i: (i, 0)),
            pl.BlockSpec((BR,), lambda i: (i,)),
            pl.BlockSpec((BR,), lambda i: (i,)),
        ],
        out_shape=[
            jax.ShapeDtypeStruct((num_rows, DP), jnp.float32),
            jax.ShapeDtypeStruct((num_rows,), jnp.float32),
            jax.ShapeDtypeStruct((num_rows,), jnp.float32),
        ],
        compiler_params=pltpu.CompilerParams(
            fuse_transposed_lhs_in_matmul=True),
    )(tt, nt, tb2, nb2)


def _body(i0_hbm, i1_hbm, comb_hbm, tbias_hbm, nbias_hbm, out_hbm,
          idx0_v, idx1_v,
          rows_t, rows_n, bval_t, bval_n,
          acc_v, all_v, out_v, shared,
          sem_e, sem_b):
    sid = lax.axis_index("s")
    base = sid * NCH

    pltpu.sync_copy(i0_hbm.at[pl.ds(base, NCH)], idx0_v)
    pltpu.sync_copy(i1_hbm.at[pl.ds(base, NCH)], idx1_v)

    emb_cps = []
    for k in range(NCH):
        emb_cps.append(pltpu.async_copy(
            comb_hbm.at[idx0_v.at[k]], rows_t.at[pl.ds(k * CHUNK, CHUNK)], sem_e))
        emb_cps.append(pltpu.async_copy(
            comb_hbm.at[idx1_v.at[k]], rows_n.at[pl.ds(k * CHUNK, CHUNK)], sem_e))

    bias_cps = []
    for k in range(NCH):
        bias_cps.append(pltpu.async_copy(
            tbias_hbm.at[idx0_v.at[k]], bval_t.at[pl.ds(k * CHUNK, CHUNK)], sem_b))
        bias_cps.append(pltpu.async_copy(
            nbias_hbm.at[idx1_v.at[k]], bval_n.at[pl.ds(k * CHUNK, CHUNK)], sem_b))

    for cp in emb_cps:
        cp.wait()

    def dot_step(b, accs):
        a0, a1, a2, a3 = accs
        a0 = a0 + rows_t[b, pl.ds(0, L)] * rows_n[b, pl.ds(D, L)]
        a1 = a1 + rows_t[b, pl.ds(L, L)] * rows_n[b, pl.ds(D + L, L)]
        a2 = a2 + rows_t[b, pl.ds(2 * L, L)] * rows_n[b, pl.ds(D + 2 * L, L)]
        a3 = a3 + rows_t[b, pl.ds(3 * L, L)] * rows_n[b, pl.ds(D + 3 * L, L)]
        return a0, a1, a2, a3

    zeros = jnp.zeros((L,), jnp.float32)
    a0, a1, a2, a3 = lax.fori_loop(0, BPW, dot_step, (zeros, zeros, zeros, zeros))
    acc = (a0 + a1) + (a2 + a3)

    acc_v[...] = acc
    pltpu.sync_copy(acc_v, shared.at[pl.ds(sid * L, L)])
    plsc.subcore_barrier()
    pltpu.sync_copy(shared, all_v)
    s = all_v[pl.ds(0, L)]
    for r in range(1, NS):
        s = s + all_v[pl.ds(r * L, L)]
    lanes = jnp.arange(L, dtype=jnp.int32)
    for off in (8, 4, 2, 1):
        perm = jnp.bitwise_xor(lanes, off)
        s = s + s.at[perm].get(mode="promise_in_bounds")
    big_s = s

    for cp in bias_cps:
        cp.wait()

    for j in range(BPW // L):
        sl = pl.ds(j * L, L)
        x = big_s + bval_t[sl] + bval_n[sl]
        out_v[sl] = 1.0 / (1.0 + jnp.exp(-x))

    pltpu.sync_copy(out_v, out_hbm.at[pl.ds(sid * BPW, BPW)])


def kernel(inputs, track_embedding, name_embedding, track_bias, name_bias):
    num_track = track_embedding.shape[0]
    i0 = inputs[:, 0].reshape(B // CHUNK, CHUNK)
    i1 = inputs[:, 1].reshape(B // CHUNK, CHUNK)
    comb, tb, nb = _relayout(
        track_embedding.T, name_embedding.T,
        track_bias.T, name_bias.T, num_track)

    mesh = plsc.VectorSubcoreMesh(
        core_axis_name="c", subcore_axis_name="s", num_cores=1)
    run = pl.kernel(
        _body,
        out_type=jax.ShapeDtypeStruct((B,), jnp.float32),
        mesh=mesh,
        compiler_params=pltpu.CompilerParams(use_tc_tiling_on_sc=True),
        scratch_types=[
            pltpu.VMEM((NCH, CHUNK), jnp.int32),
            pltpu.VMEM((NCH, CHUNK), jnp.int32),
            pltpu.VMEM((BPW, DP), jnp.float32),
            pltpu.VMEM((BPW, DP), jnp.float32),
            pltpu.VMEM((BPW,), jnp.float32),
            pltpu.VMEM((BPW,), jnp.float32),
            pltpu.VMEM((L,), jnp.float32),
            pltpu.VMEM((NS * L,), jnp.float32),
            pltpu.VMEM((BPW,), jnp.float32),
            pltpu.VMEM_SHARED((NS * L,), jnp.float32),
            pltpu.SemaphoreType.DMA,
            pltpu.SemaphoreType.DMA,
        ],
    )
    out = run(i0, i1, comb, tb, nb)
    return out.reshape(B, 1)


# speedup vs baseline: 2.4466x; 1.3780x over previous
import functools

import jax
import jax.numpy as jnp
from jax import lax
from jax.experimental import pallas as pl
from jax.experimental.pallas import tpu as pltpu
from jax.experimental.pallas import tpu_sc as plsc

L = 16
NS = 16
B = 4096
D = 64
DP = 128
BPW = B // NS
CHUNK = 128
NCH = BPW // CHUNK
BR = 16384


def _transpose_body(tt_ref, nt_ref, tb_ref, nb_ref, o_ref, tbo_ref, nbo_ref):
    o_ref[...] = jnp.concatenate([tt_ref[...], nt_ref[...]], axis=0).T
    tbo_ref[...] = tb_ref[0, :]
    nbo_ref[...] = nb_ref[0, :]


def _relayout(tt, nt, tb2, nb2, num_rows):
    grid = (pl.cdiv(num_rows, BR),)
    return pl.pallas_call(
        _transpose_body,
        grid=grid,
        in_specs=[
            pl.BlockSpec((D, BR), lambda i: (0, i)),
            pl.BlockSpec((D, BR), lambda i: (0, i)),
            pl.BlockSpec((1, BR), lambda i: (0, i)),
            pl.BlockSpec((1, BR), lambda i: (0, i)),
        ],
        out_specs=[
            pl.BlockSpec((BR, DP), lambda i: (i, 0)),
            pl.BlockSpec((BR,), lambda i: (i,)),
            pl.BlockSpec((BR,), lambda i: (i,)),
        ],
        out_shape=[
            jax.ShapeDtypeStruct((num_rows, DP), jnp.float32),
            jax.ShapeDtypeStruct((num_rows,), jnp.float32),
            jax.ShapeDtypeStruct((num_rows,), jnp.float32),
        ],
        compiler_params=pltpu.CompilerParams(
            fuse_transposed_lhs_in_matmul=True,
            vmem_limit_bytes=100 * 1024 * 1024),
    )(tt, nt, tb2, nb2)


def _body(i0_hbm, i1_hbm, comb_hbm, tbias_hbm, nbias_hbm, out_hbm,
          idx0_v, idx1_v, gidx0_v, gidx1_v,
          rows_t, rows_n, bval_t, bval_n,
          acc_v, all_v, out_v, shared,
          sem_e, sem_b):
    sid = lax.axis_index("s")
    base = sid * NCH

    pltpu.sync_copy(i0_hbm.at[pl.ds(base, NCH)], idx0_v)
    pltpu.sync_copy(i1_hbm.at[pl.ds(base, NCH)], idx1_v)

    def gidx_step(j, _):
        sl = pl.ds(j * L, L)
        for k in range(NCH):
            gidx0_v[k, sl] = idx0_v[k, sl] * 2
            gidx1_v[k, sl] = idx1_v[k, sl] * 2 + 1
        return 0

    lax.fori_loop(0, CHUNK // L, gidx_step, 0)

    emb_cps = []
    for k in range(NCH):
        emb_cps.append(pltpu.async_copy(
            comb_hbm.at[gidx0_v.at[k]], rows_t.at[pl.ds(k * CHUNK, CHUNK)], sem_e))
        emb_cps.append(pltpu.async_copy(
            comb_hbm.at[gidx1_v.at[k]], rows_n.at[pl.ds(k * CHUNK, CHUNK)], sem_e))

    bias_cps = []
    for k in range(NCH):
        bias_cps.append(pltpu.async_copy(
            tbias_hbm.at[idx0_v.at[k]], bval_t.at[pl.ds(k * CHUNK, CHUNK)], sem_b))
        bias_cps.append(pltpu.async_copy(
            nbias_hbm.at[idx1_v.at[k]], bval_n.at[pl.ds(k * CHUNK, CHUNK)], sem_b))

    for cp in emb_cps:
        cp.wait()

    def dot_step(b, accs):
        a0, a1, a2, a3 = accs
        a0 = a0 + rows_t[b, pl.ds(0, L)] * rows_n[b, pl.ds(0, L)]
        a1 = a1 + rows_t[b, pl.ds(L, L)] * rows_n[b, pl.ds(L, L)]
        a2 = a2 + rows_t[b, pl.ds(2 * L, L)] * rows_n[b, pl.ds(2 * L, L)]
        a3 = a3 + rows_t[b, pl.ds(3 * L, L)] * rows_n[b, pl.ds(3 * L, L)]
        return a0, a1, a2, a3

    zeros = jnp.zeros((L,), jnp.float32)
    a0, a1, a2, a3 = lax.fori_loop(0, BPW, dot_step, (zeros, zeros, zeros, zeros))
    acc = (a0 + a1) + (a2 + a3)

    acc_v[...] = acc
    pltpu.sync_copy(acc_v, shared.at[pl.ds(sid * L, L)])
    plsc.subcore_barrier()
    pltpu.sync_copy(shared, all_v)
    s = lax.fori_loop(
        1, NS, lambda r, a: a + all_v[pl.ds(r * L, L)], all_v[pl.ds(0, L)])
    lanes = jnp.arange(L, dtype=jnp.int32)
    for off in (8, 4, 2, 1):
        perm = jnp.bitwise_xor(lanes, off)
        s = s + s.at[perm].get(mode="promise_in_bounds")
    big_s = s

    for cp in bias_cps:
        cp.wait()

    def out_step(j, _):
        sl = pl.ds(j * L, L)
        x = big_s + bval_t[sl] + bval_n[sl]
        out_v[sl] = 1.0 / (1.0 + jnp.exp(-x))
        return 0

    lax.fori_loop(0, BPW // L, out_step, 0)

    pltpu.sync_copy(out_v, out_hbm.at[pl.ds(sid * BPW, BPW)])


def kernel(inputs, track_embedding, name_embedding, track_bias, name_bias):
    num_track = track_embedding.shape[0]
    i0 = inputs[:, 0].reshape(B // CHUNK, CHUNK)
    i1 = inputs[:, 1].reshape(B // CHUNK, CHUNK)
    comb, tb, nb = _relayout(
        track_embedding.T, name_embedding.T,
        track_bias.T, name_bias.T, num_track)
    comb = comb.reshape(2 * num_track, D)

    mesh = plsc.VectorSubcoreMesh(
        core_axis_name="c", subcore_axis_name="s", num_cores=1)
    run = pl.kernel(
        _body,
        out_type=jax.ShapeDtypeStruct((B,), jnp.float32),
        mesh=mesh,
        compiler_params=pltpu.CompilerParams(use_tc_tiling_on_sc=False),
        scratch_types=[
            pltpu.VMEM((NCH, CHUNK), jnp.int32),
            pltpu.VMEM((NCH, CHUNK), jnp.int32),
            pltpu.VMEM((NCH, CHUNK), jnp.int32),
            pltpu.VMEM((NCH, CHUNK), jnp.int32),
            pltpu.VMEM((BPW, D), jnp.float32),
            pltpu.VMEM((BPW, D), jnp.float32),
            pltpu.VMEM((BPW,), jnp.float32),
            pltpu.VMEM((BPW,), jnp.float32),
            pltpu.VMEM((L,), jnp.float32),
            pltpu.VMEM((NS * L,), jnp.float32),
            pltpu.VMEM((BPW,), jnp.float32),
            pltpu.VMEM_SHARED((NS * L,), jnp.float32),
            pltpu.SemaphoreType.DMA,
            pltpu.SemaphoreType.DMA,
        ],
    )
    out = run(i0, i1, comb, tb, nb)
    return out.reshape(B, 1)


# speedup vs baseline: 2.4552x; 1.0035x over previous
import jax
import jax.numpy as jnp
from jax import lax
from jax.experimental import pallas as pl
from jax.experimental.pallas import tpu as pltpu
from jax.experimental.pallas import tpu_sc as plsc

L = 16
NS = 16
B = 4096
D = 64
DP = 128
BPW = B // NS
CHUNK = 128
NCH = BPW // CHUNK
BR = 16384


def _transpose_body(tt_ref, nt_ref, tb_ref, nb_ref, o_ref, tbo_ref, nbo_ref):
    o_ref[...] = jnp.concatenate([tt_ref[...], nt_ref[...]], axis=0).T
    tbo_ref[...] = tb_ref[0, :]
    nbo_ref[...] = nb_ref[0, :]


def _relayout(tt, nt, tb2, nb2, num_rows):
    grid = (pl.cdiv(num_rows, BR),)
    return pl.pallas_call(
        _transpose_body,
        grid=grid,
        in_specs=[
            pl.BlockSpec((D, BR), lambda i: (0, i)),
            pl.BlockSpec((D, BR), lambda i: (0, i)),
            pl.BlockSpec((1, BR), lambda i: (0, i)),
            pl.BlockSpec((1, BR), lambda i: (0, i)),
        ],
        out_specs=[
            pl.BlockSpec((BR, DP), lambda i: (i, 0)),
            pl.BlockSpec((BR,), lambda i: (i,)),
            pl.BlockSpec((BR,), lambda i: (i,)),
        ],
        out_shape=[
            jax.ShapeDtypeStruct((num_rows, DP), jnp.float32),
            jax.ShapeDtypeStruct((num_rows,), jnp.float32),
            jax.ShapeDtypeStruct((num_rows,), jnp.float32),
        ],
        compiler_params=pltpu.CompilerParams(
            fuse_transposed_lhs_in_matmul=True,
            vmem_limit_bytes=100 * 1024 * 1024),
    )(tt, nt, tb2, nb2)


def _body(i0_hbm, i1_hbm, comb_hbm, tbias_hbm, nbias_hbm, out_hbm,
          idx0_v, idx1_v, gidx0_v, gidx1_v,
          rows_t, rows_n, bval_t, bval_n,
          acc_v, all_v, out_v, shared,
          sem_e, sem_b):
    sid = lax.axis_index("s")
    base = sid * NCH

    pltpu.sync_copy(i0_hbm.at[pl.ds(base, NCH)], idx0_v)
    pltpu.sync_copy(i1_hbm.at[pl.ds(base, NCH)], idx1_v)

    def gidx_step(j, _):
        sl = pl.ds(j * L, L)
        for k in range(NCH):
            gidx0_v[k, sl] = idx0_v[k, sl] * 2
            gidx1_v[k, sl] = idx1_v[k, sl] * 2 + 1
        return 0

    lax.fori_loop(0, CHUNK // L, gidx_step, 0)

    emb_cps = []
    for k in range(NCH):
        emb_cps.append(pltpu.async_copy(
            comb_hbm.at[gidx0_v.at[k]], rows_t.at[pl.ds(k * CHUNK, CHUNK)], sem_e))
        emb_cps.append(pltpu.async_copy(
            comb_hbm.at[gidx1_v.at[k]], rows_n.at[pl.ds(k * CHUNK, CHUNK)], sem_e))

    bias_cps = []
    for k in range(NCH):
        bias_cps.append(pltpu.async_copy(
            tbias_hbm.at[idx0_v.at[k]], bval_t.at[pl.ds(k * CHUNK, CHUNK)], sem_b))
        bias_cps.append(pltpu.async_copy(
            nbias_hbm.at[idx1_v.at[k]], bval_n.at[pl.ds(k * CHUNK, CHUNK)], sem_b))

    for cp in emb_cps:
        cp.wait()

    def dot_step(b, accs):
        a0, a1, a2, a3 = accs
        a0 = a0 + rows_t[b, pl.ds(0, L)] * rows_n[b, pl.ds(0, L)]
        a1 = a1 + rows_t[b, pl.ds(L, L)] * rows_n[b, pl.ds(L, L)]
        a2 = a2 + rows_t[b, pl.ds(2 * L, L)] * rows_n[b, pl.ds(2 * L, L)]
        a3 = a3 + rows_t[b, pl.ds(3 * L, L)] * rows_n[b, pl.ds(3 * L, L)]
        return a0, a1, a2, a3

    zeros = jnp.zeros((L,), jnp.float32)
    a0, a1, a2, a3 = lax.fori_loop(0, BPW, dot_step, (zeros, zeros, zeros, zeros))
    acc = (a0 + a1) + (a2 + a3)

    acc_v[...] = acc
    pltpu.sync_copy(acc_v, shared.at[pl.ds(sid * L, L)])
    plsc.subcore_barrier()
    pltpu.sync_copy(shared, all_v)
    s = lax.fori_loop(
        1, NS, lambda r, a: a + all_v[pl.ds(r * L, L)], all_v[pl.ds(0, L)])
    lanes = jnp.arange(L, dtype=jnp.int32)
    for off in (8, 4, 2, 1):
        perm = jnp.bitwise_xor(lanes, off)
        s = s + s.at[perm].get(mode="promise_in_bounds")
    big_s = s

    for cp in bias_cps:
        cp.wait()

    def out_step(j, _):
        sl = pl.ds(j * L, L)
        x = big_s + bval_t[sl] + bval_n[sl]
        out_v[sl] = 1.0 / (1.0 + jnp.exp(-x))
        return 0

    lax.fori_loop(0, BPW // L, out_step, 0)

    pltpu.sync_copy(out_v, out_hbm.at[pl.ds(sid * BPW, BPW)])


def kernel(inputs, track_embedding, name_embedding, track_bias, name_bias):
    num_track = track_embedding.shape[0]
    i0 = inputs[:, 0].reshape(B // CHUNK, CHUNK)
    i1 = inputs[:, 1].reshape(B // CHUNK, CHUNK)
    comb, tb, nb = _relayout(
        track_embedding.T, name_embedding.T,
        track_bias.T, name_bias.T, num_track)
    comb = comb.reshape(2 * num_track, D)

    mesh = plsc.VectorSubcoreMesh(
        core_axis_name="c", subcore_axis_name="s", num_cores=1)
    run = pl.kernel(
        _body,
        out_type=jax.ShapeDtypeStruct((B,), jnp.float32),
        mesh=mesh,
        compiler_params=pltpu.CompilerParams(use_tc_tiling_on_sc=False),
        scratch_types=[
            pltpu.VMEM((NCH, CHUNK), jnp.int32),
            pltpu.VMEM((NCH, CHUNK), jnp.int32),
            pltpu.VMEM((NCH, CHUNK), jnp.int32),
            pltpu.VMEM((NCH, CHUNK), jnp.int32),
            pltpu.VMEM((BPW, D), jnp.float32),
            pltpu.VMEM((BPW, D), jnp.float32),
            pltpu.VMEM((BPW,), jnp.float32),
            pltpu.VMEM((BPW,), jnp.float32),
            pltpu.VMEM((L,), jnp.float32),
            pltpu.VMEM((NS * L,), jnp.float32),
            pltpu.VMEM((BPW,), jnp.float32),
            pltpu.VMEM_SHARED((NS * L,), jnp.float32),
            pltpu.SemaphoreType.DMA,
            pltpu.SemaphoreType.DMA,
        ],
    )
    out = run(i0, i1, comb, tb, nb)
    return out.reshape(B, 1)
